# baseline (speedup 1.0000x reference)
import dataclasses
import functools

import jax
import jax.numpy as jnp
import numpy as np
from jax import lax
from jax.experimental import pallas as pl
from jax.experimental.pallas import tpu as pltpu
from jax.experimental.pallas import tpu_sc as plsc

N = 10000
E = 160000
D_IN = 256
D_OUT = 256
H = 128
NC = 2
NS = 16
LANES = 16
CHUNK = 128
NCH = 80
PH = NCH // 2
EPT = NCH * CHUNK
E_PAD = NS * EPT
RPT = 624
TAIL = N - NS * RPT
MBLK = 1000


def _round_bf16_bits(u):
    return u + jnp.int32(0x7FFF) + jnp.bitwise_and(lax.shift_right_logical(u, 16),
                                                   jnp.int32(1))


def _matmul_body(x_ref, wlo_ref, whi_ref, o_ref):
    dn = (((1,), (1,)), ((), ()))
    a = lax.dot_general(x_ref[...], wlo_ref[...], dimension_numbers=dn,
                        preferred_element_type=jnp.float32)
    b = lax.dot_general(x_ref[...], whi_ref[...], dimension_numbers=dn,
                        preferred_element_type=jnp.float32)
    ua = _round_bf16_bits(lax.bitcast_convert_type(a, jnp.int32))
    ub = _round_bf16_bits(lax.bitcast_convert_type(b, jnp.int32))
    o_ref[...] = jnp.bitwise_or(
        jnp.bitwise_and(lax.shift_right_logical(ua, 16), jnp.int32(0xFFFF)),
        jnp.bitwise_and(ub, jnp.int32(-65536)))


def _support_halves(x, W):
    return pl.pallas_call(
        _matmul_body,
        grid=(NC, N // MBLK),
        in_specs=[
            pl.BlockSpec((MBLK, D_IN), lambda h, i: (i, 0)),
            pl.BlockSpec((H // 2, D_IN), lambda h, i: (2 * h, 0)),
            pl.BlockSpec((H // 2, D_IN), lambda h, i: (2 * h + 1, 0)),
        ],
        out_specs=pl.BlockSpec((MBLK, H // 2),
                               lambda h, i: (h * (N // MBLK) + i, 0)),
        out_shape=jax.ShapeDtypeStruct((NC * N, H // 2), jnp.int32),
    )(x, W, W)


def _lane_bcast(v16, lane):
    idx = jnp.full((LANES, 1), lane, dtype=jnp.int32)
    return lax.gather(
        v16, idx,
        lax.GatherDimensionNumbers(
            offset_dims=(), collapsed_slice_dims=(0,), start_index_map=(0,)),
        slice_sizes=(1,),
        mode=lax.GatherScatterMode.PROMISE_IN_BOUNDS,
    )


def _sc_compiler_params():
    cp = pltpu.CompilerParams()
    if "needs_layout_passes" in pltpu.CompilerParams.__dataclass_fields__:
        cp = dataclasses.replace(cp, needs_layout_passes=False)
    if "use_tc_tiling_on_sc" in pltpu.CompilerParams.__dataclass_fields__:
        cp = dataclasses.replace(cp, use_tc_tiling_on_sc=False)
    return cp


Q = H // 2


def _aggregate(support2, row3, col3, w3):
    mesh = plsc.VectorSubcoreMesh(core_axis_name="c", subcore_axis_name="s")

    @functools.partial(
        pl.kernel,
        out_type=jax.ShapeDtypeStruct((2, NC * N, Q), jnp.float32),
        mesh=mesh,
        compiler_params=_sc_compiler_params(),
        scratch_types=[
            pltpu.VMEM_SHARED((N, Q), jnp.int32),
            pltpu.VMEM_SHARED((N, Q), jnp.float32),
            pltpu.VMEM((PH, CHUNK), jnp.int32),
            pltpu.VMEM((PH, CHUNK), jnp.int32),
            pltpu.VMEM((PH, CHUNK), jnp.float32),
            pltpu.VMEM((CHUNK, Q), jnp.int32),
            pltpu.VMEM((CHUNK, Q), jnp.int32),
            pltpu.VMEM((CHUNK, Q), jnp.float32),
            pltpu.SemaphoreType.DMA,
            pltpu.SemaphoreType.DMA,
            pltpu.SemaphoreType.DMA,
        ],
    )
    def kern(sup_hbm, row_hbm, col_hbm, w_hbm, out_hbm,
             table, acc, colv, rowv, wv, rin0, rin1, rout, sg0, sg1, ss0):
        c = lax.axis_index("c")
        s = lax.axis_index("s")

        pltpu.sync_copy(sup_hbm.at[pl.ds(c * N + s * RPT, RPT)],
                        table.at[pl.ds(s * RPT, RPT)])

        @pl.when(s == NS - 1)
        def _():
            pltpu.sync_copy(sup_hbm.at[pl.ds(c * N + NS * RPT, TAIL)],
                            table.at[pl.ds(NS * RPT, TAIL)])

        def zero_acc():
            @pl.loop(0, CHUNK)
            def _(i):
                @pl.loop(0, Q // LANES)
                def _(j):
                    rout[i, pl.ds(j * LANES, LANES)] = jnp.zeros((LANES,),
                                                                 jnp.float32)

            @pl.loop(0, RPT // CHUNK)
            def _(z):
                pltpu.sync_copy(rout, acc.at[pl.ds(s * RPT + z * CHUNK, CHUNK)])

            pltpu.sync_copy(rout.at[pl.ds(0, RPT % CHUNK)],
                            acc.at[pl.ds(s * RPT + RPT - RPT % CHUNK,
                                         RPT % CHUNK)])

            @pl.when(s == NS - 1)
            def _():
                pltpu.sync_copy(rout.at[pl.ds(0, TAIL)],
                                acc.at[pl.ds(NS * RPT, TAIL)])

        zero_acc()

        def gather_start(kk, buf, sem):
            pltpu.async_copy(table.at[colv.at[kk]], buf, sem)

        def gather_wait(kk, buf, sem):
            pltpu.make_async_copy(table.at[colv.at[kk]], buf, sem).wait()

        def scat_start(kk):
            pltpu.async_copy(rout, acc.at[rowv.at[kk]], ss0, add=True)

        def scat_wait(kk):
            pltpu.make_async_copy(rout, acc.at[rowv.at[kk]], ss0).wait()

        himask = jnp.full((LANES,), jnp.int32(-65536))

        def scale(kk, buf, q):
            qlo = q == 0
            for g in range(CHUNK // LANES):
                w16 = wv[kk, pl.ds(g * LANES, LANES)]

                @pl.loop(0, LANES, unroll=4)
                def _(e, g=g, w16=w16, buf=buf):
                    wb = _lane_bcast(w16, e)
                    eidx = g * LANES + e
                    for jb in range(Q // LANES):
                        y = buf[eidx, pl.ds(jb * LANES, LANES)]
                        lo = plsc.bitcast(lax.shift_left(y, 16), jnp.float32)
                        hi = plsc.bitcast(jnp.bitwise_and(y, himask),
                                         jnp.float32)
                        v = jnp.where(qlo, lo, hi)
                        rout[eidx, pl.ds(jb * LANES, LANES)] = v * wb

        plsc.subcore_barrier()

        @pl.loop(0, 2)
        def _(q):
            @pl.loop(0, NCH // PH)
            def _(p, q=q):
                psl = pl.ds(p * PH, PH)
                pltpu.sync_copy(col_hbm.at[s, psl], colv)
                pltpu.sync_copy(row_hbm.at[s, psl], rowv)
                pltpu.sync_copy(w_hbm.at[s, psl], wv)

                gather_start(0, rin0, sg0)

                @pl.loop(0, PH, step=2)
                def _(k, q=q):
                    gather_start(k + 1, rin1, sg1)
                    gather_wait(k, rin0, sg0)

                    @pl.when(k > 0)
                    def _():
                        scat_wait(k - 1)

                    scale(k, rin0, q)
                    scat_start(k)

                    @pl.when(k + 2 < PH)
                    def _():
                        gather_start(k + 2, rin0, sg0)

                    gather_wait(k + 1, rin1, sg1)
                    scat_wait(k)
                    scale(k + 1, rin1, q)
                    scat_start(k + 1)

                scat_wait(PH - 1)

            plsc.subcore_barrier()

            pltpu.sync_copy(acc.at[pl.ds(s * RPT, RPT)],
                            out_hbm.at[q, pl.ds(c * N + s * RPT, RPT)])

            @pl.when(s == NS - 1)
            def _():
                pltpu.sync_copy(acc.at[pl.ds(NS * RPT, TAIL)],
                                out_hbm.at[q, pl.ds(c * N + NS * RPT, TAIL)])

            @pl.when(q == 0)
            def _():
                zero_acc()
                plsc.subcore_barrier()

    return kern(support2, row3, col3, w3)


def kernel(x, edge_index, edge_weight, W):
    support2 = _support_halves(x, W)
    pad = E_PAD - E
    ipad = jnp.zeros((pad,), jnp.int32)
    row3 = jnp.concatenate([edge_index[0], ipad]).reshape(NS, NCH, CHUNK)
    col3 = jnp.concatenate([edge_index[1], ipad]).reshape(NS, NCH, CHUNK)
    w3 = jnp.concatenate([edge_weight, jnp.zeros((pad,), jnp.float32)]
                         ).reshape(NS, NCH, CHUNK)
    out4 = _aggregate(support2, row3, col3, w3)
    return (out4.reshape(2, NC, N, Q).transpose(2, 1, 0, 3)
            .reshape(N, D_OUT))


# speedup vs baseline: 2.2485x; 2.2485x over previous
import functools

import jax
import jax.numpy as jnp
from jax import lax
from jax.experimental import pallas as pl
from jax.experimental.pallas import tpu as pltpu
from jax.experimental.pallas import tpu_sc as plsc

N = 10000
E = 160000
D_IN = 256
D_OUT = 256
H = 128
NC = 2
NS = 16
LANES = 16
CHUNK = 128
NCH = 80
PH = NCH // 2
EPT = NCH * CHUNK
E_PAD = NS * EPT
RPT = 624
TAIL = N - NS * RPT
MBLK = 1000


def _matmul_body(x_ref, w_ref, o_ref):
    o_ref[...] = lax.dot_general(
        x_ref[...], w_ref[...],
        dimension_numbers=(((1,), (1,)), ((), ())),
        preferred_element_type=jnp.float32,
    )


def _support_halves(x, W):
    return pl.pallas_call(
        _matmul_body,
        grid=(NC, N // MBLK),
        in_specs=[
            pl.BlockSpec((MBLK, D_IN), lambda h, i: (i, 0)),
            pl.BlockSpec((H, D_IN), lambda h, i: (h, 0)),
        ],
        out_specs=pl.BlockSpec((MBLK, H), lambda h, i: (h * (N // MBLK) + i, 0)),
        out_shape=jax.ShapeDtypeStruct((NC * N, H), jnp.float32),
    )(x, W)


def _lane_bcast(v16, lane):
    idx = jnp.full((LANES, 1), lane, dtype=jnp.int32)
    return lax.gather(
        v16, idx,
        lax.GatherDimensionNumbers(
            offset_dims=(), collapsed_slice_dims=(0,), start_index_map=(0,)),
        slice_sizes=(1,),
        mode=lax.GatherScatterMode.PROMISE_IN_BOUNDS,
    )


def _aggregate(support2, row3, col3, w3):
    mesh = plsc.VectorSubcoreMesh(core_axis_name="c", subcore_axis_name="s")

    @functools.partial(
        pl.kernel,
        out_type=jax.ShapeDtypeStruct((NC * N, H), jnp.float32),
        mesh=mesh,
        scratch_types=[
            pltpu.VMEM_SHARED((N, H), jnp.float32),
            pltpu.VMEM((PH, CHUNK), jnp.int32),
            pltpu.VMEM((PH, CHUNK), jnp.int32),
            pltpu.VMEM((PH, CHUNK), jnp.float32),
            pltpu.VMEM((CHUNK, H), jnp.float32),
            pltpu.VMEM((CHUNK, H), jnp.float32),
            pltpu.SemaphoreType.DMA,
            pltpu.SemaphoreType.DMA,
            pltpu.SemaphoreType.DMA,
            pltpu.SemaphoreType.DMA,
        ],
    )
    def kern(sup_hbm, row_hbm, col_hbm, w_hbm, out_hbm,
             acc, colv, rowv, wv, rows0, rows1, sg0, sg1, ss0, ss1):
        c = lax.axis_index("c")
        s = lax.axis_index("s")

        @pl.loop(0, CHUNK)
        def _(i):
            @pl.loop(0, H // LANES)
            def _(j):
                rows0[i, pl.ds(j * LANES, LANES)] = jnp.zeros((LANES,), jnp.float32)

        @pl.loop(0, RPT // CHUNK)
        def _(z):
            pltpu.sync_copy(rows0, acc.at[pl.ds(s * RPT + z * CHUNK, CHUNK)])

        pltpu.sync_copy(rows0.at[pl.ds(0, RPT % CHUNK)],
                        acc.at[pl.ds(s * RPT + RPT - RPT % CHUNK, RPT % CHUNK)])

        @pl.when(s == NS - 1)
        def _():
            pltpu.sync_copy(rows0.at[pl.ds(0, TAIL)], acc.at[pl.ds(NS * RPT, TAIL)])

        base = c * N

        def gather_start(kk, buf, sem):
            pltpu.async_copy(sup_hbm.at[colv.at[kk, pl.ds(0, CHUNK // 2)]],
                             buf.at[pl.ds(0, CHUNK // 2)], sem)
            pltpu.async_copy(sup_hbm.at[colv.at[kk, pl.ds(CHUNK // 2, CHUNK // 2)]],
                             buf.at[pl.ds(CHUNK // 2, CHUNK // 2)], sem)

        def gather_wait(kk, buf, sem):
            pltpu.make_async_copy(sup_hbm.at[colv.at[kk, pl.ds(0, CHUNK // 2)]],
                                  buf.at[pl.ds(0, CHUNK // 2)], sem).wait()
            pltpu.make_async_copy(sup_hbm.at[colv.at[kk, pl.ds(CHUNK // 2, CHUNK // 2)]],
                                  buf.at[pl.ds(CHUNK // 2, CHUNK // 2)], sem).wait()

        def scat_start(kk, buf, sem):
            pltpu.async_copy(buf, acc.at[rowv.at[kk]], sem, add=True)

        def scat_wait(kk, buf, sem):
            pltpu.make_async_copy(buf, acc.at[rowv.at[kk]], sem).wait()

        def scale(kk, buf):
            for g in range(CHUNK // LANES):
                w16 = wv[kk, pl.ds(g * LANES, LANES)]

                @pl.loop(0, LANES, unroll=4)
                def _(e, g=g, w16=w16, buf=buf):
                    wb = _lane_bcast(w16, e)
                    eidx = g * LANES + e
                    for j in range(H // LANES):
                        sl = pl.ds(j * LANES, LANES)
                        buf[eidx, sl] = buf[eidx, sl] * wb

        plsc.subcore_barrier()

        for p in range(NCH // PH):
            psl = pl.ds(p * PH, PH)
            pltpu.sync_copy(col_hbm.at[s, psl], colv)
            pltpu.sync_copy(row_hbm.at[s, psl], rowv)
            pltpu.sync_copy(w_hbm.at[s, psl], wv)

            @pl.loop(0, PH)
            def _(k):
                for g in range(CHUNK // LANES):
                    sl = pl.ds(g * LANES, LANES)
                    colv[k, sl] = colv[k, sl] + jnp.full((LANES,), base, jnp.int32)

            gather_start(0, rows0, sg0)

            @pl.loop(0, PH, step=2)
            def _(k):
                @pl.when(k > 0)
                def _():
                    scat_wait(k - 1, rows1, ss1)

                gather_start(k + 1, rows1, sg1)
                gather_wait(k, rows0, sg0)
                scale(k, rows0)
                scat_start(k, rows0, ss0)

                @pl.when(k + 2 < PH)
                def _():
                    scat_wait(k, rows0, ss0)
                    gather_start(k + 2, rows0, sg0)

                gather_wait(k + 1, rows1, sg1)
                scale(k + 1, rows1)
                scat_start(k + 1, rows1, ss1)

            scat_wait(PH - 2, rows0, ss0)
            scat_wait(PH - 1, rows1, ss1)

        plsc.subcore_barrier()

        pltpu.sync_copy(acc.at[pl.ds(s * RPT, RPT)],
                        out_hbm.at[pl.ds(c * N + s * RPT, RPT)])

        @pl.when(s == NS - 1)
        def _():
            pltpu.sync_copy(acc.at[pl.ds(NS * RPT, TAIL)],
                            out_hbm.at[pl.ds(c * N + NS * RPT, TAIL)])

    return kern(support2, row3, col3, w3)


def kernel(x, edge_index, edge_weight, W):
    support2 = _support_halves(x, W)
    pad = E_PAD - E
    ipad = jnp.zeros((pad,), jnp.int32)
    row3 = jnp.concatenate([edge_index[0], ipad]).reshape(NS, NCH, CHUNK)
    col3 = jnp.concatenate([edge_index[1], ipad]).reshape(NS, NCH, CHUNK)
    w3 = jnp.concatenate([edge_weight, jnp.zeros((pad,), jnp.float32)]
                         ).reshape(NS, NCH, CHUNK)
    out2 = _aggregate(support2, row3, col3, w3)
    return out2.reshape(NC, N, H).transpose(1, 0, 2).reshape(N, D_OUT)


# speedup vs baseline: 2.2517x; 1.0014x over previous
import functools

import jax
import jax.numpy as jnp
from jax import lax
from jax.experimental import pallas as pl
from jax.experimental.pallas import tpu as pltpu
from jax.experimental.pallas import tpu_sc as plsc

N = 10000
E = 160000
D_IN = 256
D_OUT = 256
H = 128
NC = 2
NS = 16
LANES = 16
CHUNK = 128
NCH = 80
PH = NCH // 2
EPT = NCH * CHUNK
E_PAD = NS * EPT
RPT = 624
TAIL = N - NS * RPT
MBLK = 1000


def _matmul_body(x_ref, w_ref, o_ref):
    o_ref[...] = lax.dot_general(
        x_ref[...], w_ref[...],
        dimension_numbers=(((1,), (1,)), ((), ())),
        preferred_element_type=jnp.float32,
    )


def _support_halves(x, W):
    return pl.pallas_call(
        _matmul_body,
        grid=(NC, N // MBLK),
        in_specs=[
            pl.BlockSpec((MBLK, D_IN), lambda h, i: (i, 0)),
            pl.BlockSpec((H, D_IN), lambda h, i: (h, 0)),
        ],
        out_specs=pl.BlockSpec((MBLK, H), lambda h, i: (h * (N // MBLK) + i, 0)),
        out_shape=jax.ShapeDtypeStruct((NC * N, H), jnp.float32),
    )(x, W)


def _lane_bcast(v16, lane):
    idx = jnp.full((LANES, 1), lane, dtype=jnp.int32)
    return lax.gather(
        v16, idx,
        lax.GatherDimensionNumbers(
            offset_dims=(), collapsed_slice_dims=(0,), start_index_map=(0,)),
        slice_sizes=(1,),
        mode=lax.GatherScatterMode.PROMISE_IN_BOUNDS,
    )


def _aggregate(support2, row3, col3, w3):
    mesh = plsc.VectorSubcoreMesh(core_axis_name="c", subcore_axis_name="s")

    @functools.partial(
        pl.kernel,
        out_type=jax.ShapeDtypeStruct((NC * N, H), jnp.float32),
        mesh=mesh,
        scratch_types=[
            pltpu.VMEM_SHARED((N, H), jnp.float32),
            pltpu.VMEM((PH, CHUNK), jnp.int32),
            pltpu.VMEM((PH, CHUNK), jnp.int32),
            pltpu.VMEM((PH, CHUNK), jnp.float32),
            pltpu.VMEM((CHUNK, H), jnp.float32),
            pltpu.VMEM((CHUNK, H), jnp.float32),
            pltpu.SemaphoreType.DMA,
            pltpu.SemaphoreType.DMA,
            pltpu.SemaphoreType.DMA,
            pltpu.SemaphoreType.DMA,
        ],
    )
    def kern(sup_hbm, row_hbm, col_hbm, w_hbm, out_hbm,
             acc, colv, rowv, wv, rows0, rows1, sg0, sg1, ss0, ss1):
        c = lax.axis_index("c")
        s = lax.axis_index("s")
        base = c * N

        def stage(p):
            psl = pl.ds(p * PH, PH)
            pltpu.sync_copy(col_hbm.at[s, psl], colv)
            pltpu.sync_copy(row_hbm.at[s, psl], rowv)
            pltpu.sync_copy(w_hbm.at[s, psl], wv)

            @pl.loop(0, PH)
            def _(k):
                for g in range(CHUNK // LANES):
                    sl = pl.ds(g * LANES, LANES)
                    colv[k, sl] = colv[k, sl] + jnp.full((LANES,), base,
                                                         jnp.int32)

        def gather_start(kk, buf, sem):
            pltpu.async_copy(sup_hbm.at[colv.at[kk]], buf, sem)

        def gather_wait(kk, buf, sem):
            pltpu.make_async_copy(sup_hbm.at[colv.at[kk]], buf, sem).wait()

        def scat_start(kk, buf, sem):
            pltpu.async_copy(buf, acc.at[rowv.at[kk]], sem, add=True)

        def scat_wait(kk, buf, sem):
            pltpu.make_async_copy(buf, acc.at[rowv.at[kk]], sem).wait()

        def scale(kk, buf):
            for g in range(CHUNK // LANES):
                w16 = wv[kk, pl.ds(g * LANES, LANES)]

                @pl.loop(0, LANES, unroll=4)
                def _(e, g=g, w16=w16, buf=buf):
                    wb = _lane_bcast(w16, e)
                    eidx = g * LANES + e
                    for j in range(H // LANES):
                        sl = pl.ds(j * LANES, LANES)
                        buf[eidx, sl] = buf[eidx, sl] * wb

        stage(0)
        gather_start(0, rows0, sg0)

        @pl.loop(0, CHUNK)
        def _(i):
            @pl.loop(0, H // LANES)
            def _(j):
                rows1[i, pl.ds(j * LANES, LANES)] = jnp.zeros((LANES,),
                                                              jnp.float32)

        @pl.loop(0, RPT // CHUNK)
        def _(z):
            pltpu.sync_copy(rows1, acc.at[pl.ds(s * RPT + z * CHUNK, CHUNK)])

        pltpu.sync_copy(rows1.at[pl.ds(0, RPT % CHUNK)],
                        acc.at[pl.ds(s * RPT + RPT - RPT % CHUNK, RPT % CHUNK)])

        @pl.when(s == NS - 1)
        def _():
            pltpu.sync_copy(rows1.at[pl.ds(0, TAIL)],
                            acc.at[pl.ds(NS * RPT, TAIL)])

        plsc.subcore_barrier()

        for p in range(NCH // PH):
            if p > 0:
                stage(p)
                gather_start(0, rows0, sg0)

            @pl.loop(0, PH, step=2)
            def _(k):
                @pl.when(k > 0)
                def _():
                    scat_wait(k - 1, rows1, ss1)

                gather_start(k + 1, rows1, sg1)
                gather_wait(k, rows0, sg0)
                scale(k, rows0)
                scat_start(k, rows0, ss0)

                @pl.when(k + 2 < PH)
                def _():
                    scat_wait(k, rows0, ss0)
                    gather_start(k + 2, rows0, sg0)

                gather_wait(k + 1, rows1, sg1)
                scale(k + 1, rows1)
                scat_start(k + 1, rows1, ss1)

            scat_wait(PH - 2, rows0, ss0)
            scat_wait(PH - 1, rows1, ss1)

        plsc.subcore_barrier()

        pltpu.sync_copy(acc.at[pl.ds(s * RPT, RPT)],
                        out_hbm.at[pl.ds(c * N + s * RPT, RPT)])

        @pl.when(s == NS - 1)
        def _():
            pltpu.sync_copy(acc.at[pl.ds(NS * RPT, TAIL)],
                            out_hbm.at[pl.ds(c * N + NS * RPT, TAIL)])

    return kern(support2, row3, col3, w3)


def kernel(x, edge_index, edge_weight, W):
    support2 = _support_halves(x, W)
    pad = E_PAD - E
    ipad = jnp.zeros((pad,), jnp.int32)
    row3 = jnp.concatenate([edge_index[0], ipad]).reshape(NS, NCH, CHUNK)
    col3 = jnp.concatenate([edge_index[1], ipad]).reshape(NS, NCH, CHUNK)
    w3 = jnp.concatenate([edge_weight, jnp.zeros((pad,), jnp.float32)]
                         ).reshape(NS, NCH, CHUNK)
    out2 = _aggregate(support2, row3, col3, w3)
    return out2.reshape(NC, N, H).transpose(1, 0, 2).reshape(N, D_OUT)


# speedup vs baseline: 2.4993x; 1.1100x over previous
import functools

import jax
import jax.numpy as jnp
from jax import lax
from jax.experimental import pallas as pl
from jax.experimental.pallas import tpu as pltpu
from jax.experimental.pallas import tpu_sc as plsc

N = 10000
E = 160000
D_IN = 256
D_OUT = 256
H = 128
NC = 2
NS = 16
LANES = 16
CHUNK = 128
NCH = 80
PH = NCH // 2
EPT = NCH * CHUNK
E_PAD = NS * EPT
RPT = 624
TAIL = N - NS * RPT
MBLK = 1000


def _matmul_body(x_ref, w_ref, o_ref):
    o_ref[...] = lax.dot_general(
        x_ref[...], w_ref[...],
        dimension_numbers=(((1,), (1,)), ((), ())),
        preferred_element_type=jnp.float32,
    )


def _support_halves(x, W):
    return pl.pallas_call(
        _matmul_body,
        grid=(NC, N // MBLK),
        in_specs=[
            pl.BlockSpec((MBLK, D_IN), lambda h, i: (i, 0)),
            pl.BlockSpec((H, D_IN), lambda h, i: (h, 0)),
        ],
        out_specs=pl.BlockSpec((MBLK, H), lambda h, i: (h * (N // MBLK) + i, 0)),
        out_shape=jax.ShapeDtypeStruct((NC * N, H), jnp.float32),
    )(x, W)


def _lane_bcast(v16, lane):
    idx = jnp.full((LANES, 1), lane, dtype=jnp.int32)
    return lax.gather(
        v16, idx,
        lax.GatherDimensionNumbers(
            offset_dims=(), collapsed_slice_dims=(0,), start_index_map=(0,)),
        slice_sizes=(1,),
        mode=lax.GatherScatterMode.PROMISE_IN_BOUNDS,
    )


def _aggregate(support2, row3, col3, w3):
    mesh = plsc.VectorSubcoreMesh(core_axis_name="c", subcore_axis_name="s")

    @functools.partial(
        pl.kernel,
        out_type=jax.ShapeDtypeStruct((N, D_OUT), jnp.float32),
        mesh=mesh,
        scratch_types=[
            pltpu.VMEM_SHARED((N, H), jnp.float32),
            pltpu.VMEM((PH, CHUNK), jnp.int32),
            pltpu.VMEM((PH, CHUNK), jnp.int32),
            pltpu.VMEM((PH, CHUNK), jnp.float32),
            pltpu.VMEM((CHUNK, H), jnp.float32),
            pltpu.VMEM((CHUNK, H), jnp.float32),
            pltpu.SemaphoreType.DMA,
            pltpu.SemaphoreType.DMA,
            pltpu.SemaphoreType.DMA,
            pltpu.SemaphoreType.DMA,
        ],
    )
    def kern(sup_hbm, row_hbm, col_hbm, w_hbm, out_hbm,
             acc, colv, rowv, wv, rows0, rows1, sg0, sg1, ss0, ss1):
        c = lax.axis_index("c")
        s = lax.axis_index("s")
        base = c * N

        def stage(p):
            psl = pl.ds(p * PH, PH)
            pltpu.sync_copy(col_hbm.at[s, psl], colv)
            pltpu.sync_copy(row_hbm.at[s, psl], rowv)
            pltpu.sync_copy(w_hbm.at[s, psl], wv)

            @pl.loop(0, PH)
            def _(k):
                for g in range(CHUNK // LANES):
                    sl = pl.ds(g * LANES, LANES)
                    colv[k, sl] = colv[k, sl] + jnp.full((LANES,), base,
                                                         jnp.int32)

        def gather_start(kk, buf, sem):
            pltpu.async_copy(sup_hbm.at[colv.at[kk]], buf, sem)

        def gather_wait(kk, buf, sem):
            pltpu.make_async_copy(sup_hbm.at[colv.at[kk]], buf, sem).wait()

        def scat_start(kk, buf, sem):
            pltpu.async_copy(buf, acc.at[rowv.at[kk]], sem, add=True)

        def scat_wait(kk, buf, sem):
            pltpu.make_async_copy(buf, acc.at[rowv.at[kk]], sem).wait()

        def scale(kk, buf):
            for g in range(CHUNK // LANES):
                w16 = wv[kk, pl.ds(g * LANES, LANES)]

                @pl.loop(0, LANES, unroll=4)
                def _(e, g=g, w16=w16, buf=buf):
                    wb = _lane_bcast(w16, e)
                    eidx = g * LANES + e
                    for j in range(H // LANES):
                        sl = pl.ds(j * LANES, LANES)
                        buf[eidx, sl] = buf[eidx, sl] * wb

        stage(0)
        gather_start(0, rows0, sg0)

        @pl.loop(0, CHUNK)
        def _(i):
            @pl.loop(0, H // LANES)
            def _(j):
                rows1[i, pl.ds(j * LANES, LANES)] = jnp.zeros((LANES,),
                                                              jnp.float32)

        @pl.loop(0, RPT // CHUNK)
        def _(z):
            pltpu.sync_copy(rows1, acc.at[pl.ds(s * RPT + z * CHUNK, CHUNK)])

        pltpu.sync_copy(rows1.at[pl.ds(0, RPT % CHUNK)],
                        acc.at[pl.ds(s * RPT + RPT - RPT % CHUNK, RPT % CHUNK)])

        @pl.when(s == NS - 1)
        def _():
            pltpu.sync_copy(rows1.at[pl.ds(0, TAIL)],
                            acc.at[pl.ds(NS * RPT, TAIL)])

        plsc.subcore_barrier()

        for p in range(NCH // PH):
            if p > 0:
                stage(p)
                gather_start(0, rows0, sg0)

            @pl.loop(0, PH, step=2)
            def _(k):
                @pl.when(k > 0)
                def _():
                    scat_wait(k - 1, rows1, ss1)

                gather_start(k + 1, rows1, sg1)
                gather_wait(k, rows0, sg0)
                scale(k, rows0)
                scat_start(k, rows0, ss0)

                @pl.when(k + 2 < PH)
                def _():
                    scat_wait(k, rows0, ss0)
                    gather_start(k + 2, rows0, sg0)

                gather_wait(k + 1, rows1, sg1)
                scale(k + 1, rows1)
                scat_start(k + 1, rows1, ss1)

            scat_wait(PH - 2, rows0, ss0)
            scat_wait(PH - 1, rows1, ss1)

        plsc.subcore_barrier()

        pltpu.sync_copy(acc.at[pl.ds(s * RPT, RPT)],
                        out_hbm.at[pl.ds(s * RPT, RPT), pl.ds(c * H, H)])

        @pl.when(s == NS - 1)
        def _():
            pltpu.sync_copy(acc.at[pl.ds(NS * RPT, TAIL)],
                            out_hbm.at[pl.ds(NS * RPT, TAIL), pl.ds(c * H, H)])

    return kern(support2, row3, col3, w3)


def kernel(x, edge_index, edge_weight, W):
    support2 = _support_halves(x, W)
    pad = E_PAD - E
    ipad = jnp.zeros((pad,), jnp.int32)
    row3 = jnp.concatenate([edge_index[0], ipad]).reshape(NS, NCH, CHUNK)
    col3 = jnp.concatenate([edge_index[1], ipad]).reshape(NS, NCH, CHUNK)
    w3 = jnp.concatenate([edge_weight, jnp.zeros((pad,), jnp.float32)]
                         ).reshape(NS, NCH, CHUNK)
    return _aggregate(support2, row3, col3, w3)
